# baseline (speedup 1.0000x reference)
import jax
import jax.numpy as jnp
from jax import lax
from jax.experimental import pallas as pl
from jax.experimental.pallas import tpu as pltpu
from jax.experimental.pallas import tpu_sc as plsc

NC = 2
NS = 16
NW = NC * NS
CH = 192


def _make_sc_agg(n_nodes, d, n_edges, with_counts):
    epw = n_edges // NW
    iters = epw // CH
    rpt = ((n_nodes + NS - 1) // NS + 7) // 8 * 8
    last = n_nodes - rpt * (NS - 1)
    n_pad = rpt * NS

    assert epw % CH == 0 and (epw // CH) % 2 == 0

    out_types = [jax.ShapeDtypeStruct((NC, n_nodes, d), jnp.float32)]
    scratch = [
        pltpu.VMEM((CH,), jnp.int32),
        pltpu.VMEM((CH,), jnp.int32),
        pltpu.VMEM((CH, d), jnp.float32),
        pltpu.VMEM((CH,), jnp.int32),
        pltpu.VMEM((CH,), jnp.int32),
        pltpu.VMEM((CH, d), jnp.float32),
        pltpu.VMEM_SHARED((n_pad, d), jnp.float32),
        pltpu.SemaphoreType.DMA,
        pltpu.SemaphoreType.DMA,
    ]
    if with_counts:
        out_types.append(jax.ShapeDtypeStruct((NC, n_nodes, d), jnp.float32))

    mesh = plsc.VectorSubcoreMesh(core_axis_name="c", subcore_axis_name="s",
                                  num_cores=NC, num_subcores=NS)

    def body(h_hbm, src_hbm, dst_hbm, zrow_hbm, ones_hbm, *rest):
        if with_counts:
            (out_hbm, cnt_hbm, idx_s0, idx_d0, rows0,
             idx_s1, idx_d1, rows1, acc, sem0, sem1) = rest
        else:
            (out_hbm, idx_s0, idx_d0, rows0,
             idx_s1, idx_d1, rows1, acc, sem0, sem1) = rest
        c = lax.axis_index("c")
        s = lax.axis_index("s")
        wid = c * NS + s
        base = wid * epw
        half = iters // 2

        pltpu.sync_copy(zrow_hbm, acc.at[pl.ds(s * rpt, rpt)])
        plsc.subcore_barrier()

        def step(i, carry):
            o = base + i * CH
            pltpu.sync_copy(src_hbm.at[pl.ds(o, CH)], idx_s0)
            pltpu.sync_copy(dst_hbm.at[pl.ds(o, CH)], idx_d0)
            pltpu.async_copy(h_hbm.at[idx_s0], rows0, sem0).wait()
            pltpu.sync_copy(rows0, acc.at[idx_d0], add=True)
            return carry

        lax.fori_loop(0, iters, step, 0)
        plsc.subcore_barrier()

        def writeout(dst_ref):
            @pl.when(s < NS - 1)
            def _full():
                pltpu.sync_copy(acc.at[pl.ds(s * rpt, rpt)],
                                dst_ref.at[c, pl.ds(s * rpt, rpt)])

            @pl.when(s == NS - 1)
            def _tail():
                pltpu.sync_copy(acc.at[pl.ds((NS - 1) * rpt, last)],
                                dst_ref.at[c, pl.ds((NS - 1) * rpt, last)])

        writeout(out_hbm)

        if with_counts:
            pltpu.sync_copy(zrow_hbm, acc.at[pl.ds(s * rpt, rpt)])
            pltpu.sync_copy(ones_hbm, rows0)
            plsc.subcore_barrier()
            pltpu.sync_copy(dst_hbm.at[pl.ds(base, CH)], idx_d0)

            def cpair(g, carry):
                pltpu.async_copy(rows0, acc.at[idx_d0], sem0, add=True)

                @pl.when(g > 0)
                def _w1():
                    pltpu.make_async_copy(rows0, acc.at[idx_d1], sem1).wait()

                pltpu.sync_copy(dst_hbm.at[pl.ds(base + (2 * g + 1) * CH, CH)],
                                idx_d1)
                pltpu.async_copy(rows0, acc.at[idx_d1], sem1, add=True)
                pltpu.make_async_copy(rows0, acc.at[idx_d0], sem0).wait()

                @pl.when(g < half - 1)
                def _ld():
                    pltpu.sync_copy(
                        dst_hbm.at[pl.ds(base + (2 * g + 2) * CH, CH)], idx_d0)

                return carry

            lax.fori_loop(0, half, cpair, 0)
            pltpu.make_async_copy(rows0, acc.at[idx_d1], sem1).wait()
            plsc.subcore_barrier()
            writeout(cnt_hbm)

    return pl.kernel(body, out_type=tuple(out_types), mesh=mesh,
                     scratch_types=scratch)


def _dense_body(p_ref, c_ref, h_ref, wl_ref, wr_ref, b_ref, o_ref):
    cnt = c_ref[0, :, 0:1] + c_ref[1, :, 0:1]
    sm = p_ref[0] + p_ref[1]
    mean = sm / jnp.maximum(cnt, 1.0)
    acc = jnp.dot(mean, wl_ref[...], preferred_element_type=jnp.float32)
    acc = acc + jnp.dot(h_ref[...], wr_ref[...], preferred_element_type=jnp.float32)
    o_ref[...] = jnp.maximum(acc + b_ref[...], 0.0)


def _dense(p, cnt, h, wl_t, wr_t, b):
    n, d = h.shape
    bn = 1000
    return pl.pallas_call(
        _dense_body,
        grid=(n // bn,),
        in_specs=[
            pl.BlockSpec((NC, bn, d), lambda i: (0, i, 0)),
            pl.BlockSpec((NC, bn, d), lambda i: (0, i, 0)),
            pl.BlockSpec((bn, d), lambda i: (i, 0)),
            pl.BlockSpec((d, d), lambda i: (0, 0)),
            pl.BlockSpec((d, d), lambda i: (0, 0)),
            pl.BlockSpec((1, d), lambda i: (0, 0)),
        ],
        out_specs=pl.BlockSpec((bn, d), lambda i: (i, 0)),
        out_shape=jax.ShapeDtypeStruct((n, d), jnp.float32),
    )(p, cnt, h, wl_t, wr_t, b)


def kernel(x, edge_index, W1_l, b1_l, W1_r, W2_l, b2_l, W2_r):
    n, d = x.shape
    e = edge_index.shape[1]
    rpt = ((n + NS - 1) // NS + 7) // 8 * 8
    n_pad = rpt * NS
    zrow = jnp.zeros((rpt, d), jnp.float32)
    ones = jnp.ones((CH, d), jnp.float32)

    step = NW * 2 * CH
    e_pad = (e + step - 1) // step * step
    src = jnp.concatenate([edge_index[0],
                           jnp.zeros((e_pad - e,), jnp.int32)])
    dst = jnp.concatenate([edge_index[1],
                           jnp.full((e_pad - e,), n_pad - 1, jnp.int32)])

    agg1 = _make_sc_agg(n, d, e_pad, True)
    agg2 = _make_sc_agg(n, d, e_pad, False)

    p1, cnt = agg1(x, src, dst, zrow, ones)
    h1 = _dense(p1, cnt, x, W1_l.T, W1_r.T, b1_l[None, :])
    p2 = agg2(h1, src, dst, zrow, ones)
    if isinstance(p2, (tuple, list)):
        p2 = p2[0]
    h2 = _dense(p2, cnt, h1, W2_l.T, W2_r.T, b2_l[None, :])
    return h2


# speedup vs baseline: 1.0005x; 1.0005x over previous
import jax
import jax.numpy as jnp
from jax import lax
from jax.experimental import pallas as pl
from jax.experimental.pallas import tpu as pltpu
from jax.experimental.pallas import tpu_sc as plsc

NC = 2
NS = 16
NW = NC * NS
CH = 192


def _make_sc_agg(n_nodes, d, n_edges, with_counts):
    epw = n_edges // NW
    iters = epw // CH
    rpt = ((n_nodes + NS - 1) // NS + 7) // 8 * 8
    last = n_nodes - rpt * (NS - 1)
    n_pad = rpt * NS

    assert epw % CH == 0 and (epw // CH) % 2 == 0

    out_types = [jax.ShapeDtypeStruct((NC, n_nodes, d), jnp.float32)]
    scratch = [
        pltpu.VMEM((CH,), jnp.int32),
        pltpu.VMEM((CH,), jnp.int32),
        pltpu.VMEM((CH, d), jnp.float32),
        pltpu.VMEM((CH,), jnp.int32),
        pltpu.VMEM((CH,), jnp.int32),
        pltpu.VMEM((CH, d), jnp.float32),
        pltpu.VMEM_SHARED((n_pad, d), jnp.float32),
        pltpu.SemaphoreType.DMA,
        pltpu.SemaphoreType.DMA,
    ]
    if with_counts:
        out_types.append(jax.ShapeDtypeStruct((NC, n_nodes, d), jnp.float32))

    mesh = plsc.VectorSubcoreMesh(core_axis_name="c", subcore_axis_name="s",
                                  num_cores=NC, num_subcores=NS)

    def body(h_hbm, src_hbm, dst_hbm, zrow_hbm, ones_hbm, *rest):
        if with_counts:
            (out_hbm, cnt_hbm, idx_s0, idx_d0, rows0,
             idx_s1, idx_d1, rows1, acc, sem0, sem1) = rest
        else:
            (out_hbm, idx_s0, idx_d0, rows0,
             idx_s1, idx_d1, rows1, acc, sem0, sem1) = rest
        c = lax.axis_index("c")
        s = lax.axis_index("s")
        wid = c * NS + s
        base = wid * epw
        half = iters // 2

        pltpu.sync_copy(zrow_hbm, acc.at[pl.ds(s * rpt, rpt)])
        plsc.subcore_barrier()

        def step(i, carry):
            o = base + i * CH
            pltpu.sync_copy(src_hbm.at[pl.ds(o, CH)], idx_s0)
            pltpu.sync_copy(dst_hbm.at[pl.ds(o, CH)], idx_d0)
            pltpu.async_copy(h_hbm.at[idx_s0], rows0, sem0).wait()
            pltpu.sync_copy(rows0, acc.at[idx_d0], add=True)
            return carry

        lax.fori_loop(0, iters, step, 0)
        plsc.subcore_barrier()

        def writeout(dst_ref):
            @pl.when(s < NS - 1)
            def _full():
                pltpu.sync_copy(acc.at[pl.ds(s * rpt, rpt)],
                                dst_ref.at[c, pl.ds(s * rpt, rpt)])

            @pl.when(s == NS - 1)
            def _tail():
                pltpu.sync_copy(acc.at[pl.ds((NS - 1) * rpt, last)],
                                dst_ref.at[c, pl.ds((NS - 1) * rpt, last)])

        writeout(out_hbm)

        if with_counts:
            pltpu.sync_copy(zrow_hbm, acc.at[pl.ds(s * rpt, rpt)])
            pltpu.sync_copy(ones_hbm, rows0)
            plsc.subcore_barrier()
            pltpu.sync_copy(dst_hbm.at[pl.ds(base, CH)], idx_d0)

            def cpair(g, carry):
                pltpu.async_copy(rows0, acc.at[idx_d0], sem0, add=True)

                @pl.when(g > 0)
                def _w1():
                    pltpu.make_async_copy(rows0, acc.at[idx_d1], sem1).wait()

                pltpu.sync_copy(dst_hbm.at[pl.ds(base + (2 * g + 1) * CH, CH)],
                                idx_d1)
                pltpu.async_copy(rows0, acc.at[idx_d1], sem1, add=True)
                pltpu.make_async_copy(rows0, acc.at[idx_d0], sem0).wait()

                @pl.when(g < half - 1)
                def _ld():
                    pltpu.sync_copy(
                        dst_hbm.at[pl.ds(base + (2 * g + 2) * CH, CH)], idx_d0)

                return carry

            lax.fori_loop(0, half, cpair, 0)
            pltpu.make_async_copy(rows0, acc.at[idx_d1], sem1).wait()
            plsc.subcore_barrier()
            writeout(cnt_hbm)

    return pl.kernel(body, out_type=tuple(out_types), mesh=mesh,
                     scratch_types=scratch)


def _dense_body(p_ref, c_ref, h_ref, wl_ref, wr_ref, b_ref, o_ref):
    cnt = c_ref[0, :, 0:1] + c_ref[1, :, 0:1]
    sm = p_ref[0] + p_ref[1]
    mean = sm / jnp.maximum(cnt, 1.0)
    acc = jnp.dot(mean, wl_ref[...], preferred_element_type=jnp.float32)
    acc = acc + jnp.dot(h_ref[...], wr_ref[...], preferred_element_type=jnp.float32)
    o_ref[...] = jnp.maximum(acc + b_ref[...], 0.0)


def _dense(p, cnt, h, wl_t, wr_t, b):
    n, d = h.shape
    bn = 1000
    return pl.pallas_call(
        _dense_body,
        grid=(n // bn,),
        in_specs=[
            pl.BlockSpec((NC, bn, d), lambda i: (0, i, 0)),
            pl.BlockSpec((NC, bn, d), lambda i: (0, i, 0)),
            pl.BlockSpec((bn, d), lambda i: (i, 0)),
            pl.BlockSpec((d, d), lambda i: (0, 0)),
            pl.BlockSpec((d, d), lambda i: (0, 0)),
            pl.BlockSpec((1, d), lambda i: (0, 0)),
        ],
        out_specs=pl.BlockSpec((bn, d), lambda i: (i, 0)),
        out_shape=jax.ShapeDtypeStruct((n, d), jnp.float32),
    )(p, cnt, h, wl_t, wr_t, b)


def kernel(x, edge_index, W1_l, b1_l, W1_r, W2_l, b2_l, W2_r):
    n, d = x.shape
    e = edge_index.shape[1]
    rpt = ((n + NS - 1) // NS + 7) // 8 * 8
    n_pad = rpt * NS
    zrow = jnp.zeros((rpt, d), jnp.float32)
    ones = jnp.ones((CH, d), jnp.float32)

    step = NW * 2 * CH
    e_pad = (e + step - 1) // step * step
    src = jnp.concatenate([edge_index[0],
                           jnp.zeros((e_pad - e,), jnp.int32)])
    dst = jnp.concatenate([edge_index[1],
                           n + jnp.arange(e_pad - e, dtype=jnp.int32)
                           % (n_pad - n)])

    agg1 = _make_sc_agg(n, d, e_pad, True)
    agg2 = _make_sc_agg(n, d, e_pad, False)

    p1, cnt = agg1(x, src, dst, zrow, ones)
    h1 = _dense(p1, cnt, x, W1_l.T, W1_r.T, b1_l[None, :])
    p2 = agg2(h1, src, dst, zrow, ones)
    if isinstance(p2, (tuple, list)):
        p2 = p2[0]
    h2 = _dense(p2, cnt, h1, W2_l.T, W2_r.T, b2_l[None, :])
    return h2


# speedup vs baseline: 3.0685x; 3.0671x over previous
import jax
import jax.numpy as jnp
from jax import lax
from jax.experimental import pallas as pl
from jax.experimental.pallas import tpu as pltpu
from jax.experimental.pallas import tpu_sc as plsc

NC = 2
NS = 16
NW = NC * NS
CH = 200


def _make_sc_agg(n_nodes, d, n_edges, with_counts):
    epw = n_edges // NW
    iters = epw // CH
    rpt = ((n_nodes + NS - 1) // NS + 7) // 8 * 8
    last = n_nodes - rpt * (NS - 1)
    n_pad = rpt * NS

    assert epw % CH == 0

    out_types = [jax.ShapeDtypeStruct((NC, n_nodes, d), jnp.float32)]
    scratch = [
        pltpu.VMEM((CH,), jnp.int32),
        pltpu.VMEM((CH,), jnp.int32),
        pltpu.VMEM((CH, d), jnp.float32),
        pltpu.VMEM_SHARED((n_pad, d), jnp.float32),
        pltpu.SemaphoreType.DMA,
    ]
    if with_counts:
        out_types.append(jax.ShapeDtypeStruct((NC, n_nodes, d), jnp.float32))

    mesh = plsc.VectorSubcoreMesh(core_axis_name="c", subcore_axis_name="s",
                                  num_cores=NC, num_subcores=NS)

    def body(h_hbm, src_hbm, dst_hbm, zrow_hbm, ones_hbm, *rest):
        if with_counts:
            out_hbm, cnt_hbm, idx_s0, idx_d0, rows0, acc, sem0 = rest
        else:
            out_hbm, idx_s0, idx_d0, rows0, acc, sem0 = rest
        c = lax.axis_index("c")
        s = lax.axis_index("s")
        wid = c * NS + s
        base = wid * epw

        pltpu.sync_copy(zrow_hbm, acc.at[pl.ds(s * rpt, rpt)])
        plsc.subcore_barrier()

        def step(i, carry):
            o = base + i * CH
            pltpu.sync_copy(src_hbm.at[pl.ds(o, CH)], idx_s0)
            pltpu.sync_copy(dst_hbm.at[pl.ds(o, CH)], idx_d0)
            pltpu.async_copy(h_hbm.at[idx_s0], rows0, sem0).wait()
            pltpu.sync_copy(rows0, acc.at[idx_d0], add=True)
            return carry

        lax.fori_loop(0, iters, step, 0)
        plsc.subcore_barrier()

        def writeout(dst_ref):
            @pl.when(s < NS - 1)
            def _full():
                pltpu.sync_copy(acc.at[pl.ds(s * rpt, rpt)],
                                dst_ref.at[c, pl.ds(s * rpt, rpt)])

            @pl.when(s == NS - 1)
            def _tail():
                pltpu.sync_copy(acc.at[pl.ds((NS - 1) * rpt, last)],
                                dst_ref.at[c, pl.ds((NS - 1) * rpt, last)])

        writeout(out_hbm)

        if with_counts:
            pltpu.sync_copy(zrow_hbm, acc.at[pl.ds(s * rpt, rpt)])
            pltpu.sync_copy(ones_hbm, rows0)
            plsc.subcore_barrier()

            def step_c(i, carry):
                o = base + i * CH
                pltpu.sync_copy(dst_hbm.at[pl.ds(o, CH)], idx_d0)
                pltpu.sync_copy(rows0, acc.at[idx_d0], add=True)
                return carry

            lax.fori_loop(0, iters, step_c, 0)
            plsc.subcore_barrier()
            writeout(cnt_hbm)

    return pl.kernel(body, out_type=tuple(out_types), mesh=mesh,
                     scratch_types=scratch)


def _dense_body(p_ref, c_ref, h_ref, wl_ref, wr_ref, b_ref, o_ref):
    cnt = c_ref[0, :, 0:1] + c_ref[1, :, 0:1]
    sm = p_ref[0] + p_ref[1]
    mean = sm / jnp.maximum(cnt, 1.0)
    acc = jnp.dot(mean, wl_ref[...], preferred_element_type=jnp.float32)
    acc = acc + jnp.dot(h_ref[...], wr_ref[...], preferred_element_type=jnp.float32)
    o_ref[...] = jnp.maximum(acc + b_ref[...], 0.0)


def _dense(p, cnt, h, wl_t, wr_t, b):
    n, d = h.shape
    bn = 1000
    return pl.pallas_call(
        _dense_body,
        grid=(n // bn,),
        in_specs=[
            pl.BlockSpec((NC, bn, d), lambda i: (0, i, 0)),
            pl.BlockSpec((NC, bn, d), lambda i: (0, i, 0)),
            pl.BlockSpec((bn, d), lambda i: (i, 0)),
            pl.BlockSpec((d, d), lambda i: (0, 0)),
            pl.BlockSpec((d, d), lambda i: (0, 0)),
            pl.BlockSpec((1, d), lambda i: (0, 0)),
        ],
        out_specs=pl.BlockSpec((bn, d), lambda i: (i, 0)),
        out_shape=jax.ShapeDtypeStruct((n, d), jnp.float32),
    )(p, cnt, h, wl_t, wr_t, b)


def kernel(x, edge_index, W1_l, b1_l, W1_r, W2_l, b2_l, W2_r):
    n, d = x.shape
    e = edge_index.shape[1]
    rpt = ((n + NS - 1) // NS + 7) // 8 * 8
    n_pad = rpt * NS
    zrow = jnp.zeros((rpt, d), jnp.float32)
    ones = jnp.ones((CH, d), jnp.float32)

    src = edge_index[0]
    dst = edge_index[1]

    agg1 = _make_sc_agg(n, d, e, True)
    agg2 = _make_sc_agg(n, d, e, False)

    p1, cnt = agg1(x, src, dst, zrow, ones)
    h1 = _dense(p1, cnt, x, W1_l.T, W1_r.T, b1_l[None, :])
    p2 = agg2(h1, src, dst, zrow, ones)
    if isinstance(p2, (tuple, list)):
        p2 = p2[0]
    h2 = _dense(p2, cnt, h1, W2_l.T, W2_r.T, b2_l[None, :])
    return h2


# speedup vs baseline: 3.0686x; 1.0000x over previous
import jax
import jax.numpy as jnp
from jax import lax
from jax.experimental import pallas as pl
from jax.experimental.pallas import tpu as pltpu
from jax.experimental.pallas import tpu_sc as plsc

NC = 2
NS = 16
NW = NC * NS
CH = 200


def _make_sc_agg(n_nodes, d, n_edges, with_counts):
    epw = n_edges // NW
    iters = epw // CH
    rpt = ((n_nodes + NS - 1) // NS + 7) // 8 * 8
    last = n_nodes - rpt * (NS - 1)
    n_pad = rpt * NS

    assert epw % CH == 0

    out_types = [jax.ShapeDtypeStruct((NC, n_nodes, d), jnp.float32)]
    scratch = [
        pltpu.VMEM((CH,), jnp.int32),
        pltpu.VMEM((CH,), jnp.int32),
        pltpu.VMEM((CH, d), jnp.float32),
        pltpu.VMEM_SHARED((n_pad, d), jnp.float32),
        pltpu.SemaphoreType.DMA,
    ]
    if with_counts:
        out_types.append(jax.ShapeDtypeStruct((NC, n_nodes, d), jnp.float32))

    mesh = plsc.VectorSubcoreMesh(core_axis_name="c", subcore_axis_name="s",
                                  num_cores=NC, num_subcores=NS)

    def body(h_hbm, src_hbm, dst_hbm, zrow_hbm, ones_hbm, *rest):
        if with_counts:
            out_hbm, cnt_hbm, idx_s0, idx_d0, rows0, acc, sem0 = rest
        else:
            out_hbm, idx_s0, idx_d0, rows0, acc, sem0 = rest
        c = lax.axis_index("c")
        s = lax.axis_index("s")
        wid = c * NS + s
        base = wid * epw

        pltpu.sync_copy(zrow_hbm, acc.at[pl.ds(s * rpt, rpt)])
        plsc.subcore_barrier()

        def step(i, carry):
            o = base + i * CH
            pltpu.sync_copy(src_hbm.at[pl.ds(o, CH)], idx_s0)
            pltpu.sync_copy(dst_hbm.at[pl.ds(o, CH)], idx_d0)
            pltpu.async_copy(h_hbm.at[idx_s0], rows0, sem0).wait()
            pltpu.sync_copy(rows0, acc.at[idx_d0], add=True)
            return carry

        lax.fori_loop(0, iters, step, 0)
        plsc.subcore_barrier()

        def writeout(dst_ref):
            @pl.when(s < NS - 1)
            def _full():
                pltpu.sync_copy(acc.at[pl.ds(s * rpt, rpt)],
                                dst_ref.at[c, pl.ds(s * rpt, rpt)])

            @pl.when(s == NS - 1)
            def _tail():
                pltpu.sync_copy(acc.at[pl.ds((NS - 1) * rpt, last)],
                                dst_ref.at[c, pl.ds((NS - 1) * rpt, last)])

        writeout(out_hbm)

        if with_counts:
            pltpu.sync_copy(zrow_hbm, acc.at[pl.ds(s * rpt, rpt)])
            pltpu.sync_copy(ones_hbm, rows0)
            plsc.subcore_barrier()

            def step_c(i, carry):
                o = base + i * CH
                pltpu.sync_copy(dst_hbm.at[pl.ds(o, CH)], idx_d0)
                pltpu.sync_copy(rows0, acc.at[idx_d0], add=True)
                return carry

            lax.fori_loop(0, iters, step_c, 0)
            plsc.subcore_barrier()
            writeout(cnt_hbm)

    return pl.kernel(body, out_type=tuple(out_types), mesh=mesh,
                     scratch_types=scratch)


def _dense_body(p_ref, c_ref, h_ref, wl_ref, wr_ref, b_ref, o_ref):
    cnt = c_ref[0, :, 0:1] + c_ref[1, :, 0:1]
    sm = p_ref[0] + p_ref[1]
    mean = sm / jnp.maximum(cnt, 1.0)
    acc = jnp.dot(mean, wl_ref[...], preferred_element_type=jnp.float32)
    acc = acc + jnp.dot(h_ref[...], wr_ref[...], preferred_element_type=jnp.float32)
    o_ref[...] = jnp.maximum(acc + b_ref[...], 0.0)


def _dense(p, cnt, h, wl_t, wr_t, b):
    n, d = h.shape
    bn = 1000
    return pl.pallas_call(
        _dense_body,
        grid=(n // bn,),
        in_specs=[
            pl.BlockSpec((NC, bn, d), lambda i: (0, i, 0)),
            pl.BlockSpec((NC, bn, d), lambda i: (0, i, 0)),
            pl.BlockSpec((bn, d), lambda i: (i, 0)),
            pl.BlockSpec((d, d), lambda i: (0, 0)),
            pl.BlockSpec((d, d), lambda i: (0, 0)),
            pl.BlockSpec((1, d), lambda i: (0, 0)),
        ],
        out_specs=pl.BlockSpec((bn, d), lambda i: (i, 0)),
        out_shape=jax.ShapeDtypeStruct((n, d), jnp.float32),
    )(p, cnt, h, wl_t, wr_t, b)


def kernel(x, edge_index, W1_l, b1_l, W1_r, W2_l, b2_l, W2_r):
    n, d = x.shape
    e = edge_index.shape[1]
    rpt = ((n + NS - 1) // NS + 7) // 8 * 8
    n_pad = rpt * NS
    zrow = jnp.zeros((rpt, d), jnp.float32)
    ones = jnp.ones((CH, d), jnp.float32)

    step = NW * 2 * CH
    e_pad = (e + step - 1) // step * step
    src = jnp.concatenate([edge_index[0],
                           jnp.zeros((e_pad - e,), jnp.int32)])
    dst = jnp.concatenate([edge_index[1],
                           n + jnp.arange(e_pad - e, dtype=jnp.int32)
                           % (n_pad - n)])

    agg1 = _make_sc_agg(n, d, e_pad, True)
    agg2 = _make_sc_agg(n, d, e_pad, False)

    p1, cnt = agg1(x, src, dst, zrow, ones)
    h1 = _dense(p1, cnt, x, W1_l.T, W1_r.T, b1_l[None, :])
    p2 = agg2(h1, src, dst, zrow, ones)
    if isinstance(p2, (tuple, list)):
        p2 = p2[0]
    h2 = _dense(p2, cnt, h1, W2_l.T, W2_r.T, b2_l[None, :])
    return h2
